# baseline (speedup 1.0000x reference)
import jax
import jax.numpy as jnp
from jax import lax
from jax.experimental import pallas as pl
from jax.experimental.pallas import tpu as pltpu
from jax.experimental.pallas import tpu_sc as plsc

_NC = 2
_NS = 16
_LANES = 16
_CHUNK = 80
_NPAD = 10112


def _emb_body(x_ref, w_ref, b_ref, o_ref):
    o_ref[...] = (
        jnp.dot(x_ref[...], w_ref[...], preferred_element_type=jnp.float32)
        + b_ref[...]
    )


def _gate_body(ea_ref, w_ref, o_ref):
    ea = ea_ref[...]
    w = w_ref[...]
    z = (ea[:, 0:1] * w[0:1, :] + ea[:, 1:2] * w[1:2, :]
         + ea[:, 2:3] * w[2:3, :] + ea[:, 3:4] * w[3:4, :])
    gate = 1.0 / (1.0 + jnp.exp(-z))
    d2 = gate.shape[1] // 2
    a = gate[:, :d2].astype(jnp.bfloat16).astype(jnp.float32)
    b = gate[:, d2:].astype(jnp.bfloat16).astype(jnp.float32)
    ai = lax.bitcast_convert_type(a, jnp.int32)
    bi = lax.bitcast_convert_type(b, jnp.int32)
    o_ref[...] = jnp.bitwise_or(lax.shift_right_logical(ai, 16), bi)


def _edge_sc_body(h_hbm, src_hbm, dst_hbm, gate_hbm, out_hbm,
                  srcb_ref, dstb_ref, rows_ref, gbuf_ref,
                  agg_ref, sg0, sg1, sa0, sa1, ss0, ss1,
                  si0, si1, si2, si3):
    N, D = h_hbm.shape
    E = src_hbm.shape[0]
    nj = D // _LANES
    cid = lax.axis_index("c")
    sid = lax.axis_index("s")
    rows_per_tile = _NPAD // _NS
    tile_id = cid * _NS + sid
    edges_per_tile = E // (_NC * _NS)
    nchunks = edges_per_tile // _CHUNK
    ebase = tile_id * edges_per_tile
    sg = (sg0, sg1)
    sa = (sa0, sa1)
    ss = (ss0, ss1)
    si = (si0, si1, si2, si3)

    def _idx_slices(c):
        return (src_hbm.at[pl.ds(ebase + c * _CHUNK, _CHUNK)],
                dst_hbm.at[pl.ds(ebase + c * _CHUNK, _CHUNK)])

    def _issue_idx(c, slot):
        s, d = _idx_slices(c)
        pltpu.async_copy(s, srcb_ref.at[slot], si[slot])
        pltpu.async_copy(d, dstb_ref.at[slot], si[slot])

    def _wait_idx(slot):
        pltpu.make_async_copy(src_hbm.at[pl.ds(0, _CHUNK)],
                              srcb_ref.at[slot], si[slot]).wait()
        pltpu.make_async_copy(dst_hbm.at[pl.ds(0, _CHUNK)],
                              dstb_ref.at[slot], si[slot]).wait()

    def _sync_idx(c, slot):
        s, d = _idx_slices(c)
        pltpu.sync_copy(s, srcb_ref.at[slot])
        pltpu.sync_copy(d, dstb_ref.at[slot])

    zb = rows_ref.at[0]
    zeros = jnp.zeros((_LANES,), jnp.float32)

    def _zrow(r, carry):
        for j in range(nj):
            zb[r, pl.ds(j * _LANES, _LANES)] = zeros
        return carry

    lax.fori_loop(0, _CHUNK, _zrow, 0)
    nfull = rows_per_tile // _CHUNK
    tail = rows_per_tile - nfull * _CHUNK
    for t in range(nfull):
        r0 = sid * rows_per_tile + t * _CHUNK
        pltpu.sync_copy(zb, agg_ref.at[pl.ds(r0, _CHUNK), :])
    pltpu.sync_copy(zb.at[pl.ds(0, tail), :],
                    agg_ref.at[pl.ds(sid * rows_per_tile + nfull * _CHUNK,
                                     tail), :])
    plsc.subcore_barrier()

    def _issue_gather(c, islot, slot):
        pltpu.async_copy(h_hbm.at[srcb_ref.at[islot]],
                         rows_ref.at[slot], sg[slot])
        pltpu.async_copy(
            gate_hbm.at[pl.ds(ebase + c * _CHUNK, _CHUNK), :],
            gbuf_ref.at[slot], sa[slot])

    def _wait_gather(slot):
        pltpu.make_async_copy(h_hbm.at[pl.ds(0, _CHUNK)],
                              rows_ref.at[slot], sg[slot]).wait()
        pltpu.make_async_copy(gate_hbm.at[pl.ds(0, _CHUNK), :],
                              gbuf_ref.at[slot], sa[slot]).wait()

    def _wait_scatter(slot):
        pltpu.make_async_copy(rows_ref.at[slot],
                              agg_ref.at[pl.ds(0, _CHUNK), :],
                              ss[slot]).wait()

    def _compute(slot):
        rr = rows_ref.at[slot]
        gg = gbuf_ref.at[slot]
        himask = jnp.int32(-65536)
        half = (nj // 2) * _LANES

        @plsc.parallel_loop(0, _CHUNK, unroll=2)
        def _e(e):
            for t in range(nj // 2):
                gv = gg[e, pl.ds(t * _LANES, _LANES)]
                ge = lax.bitcast_convert_type(jnp.left_shift(gv, 16),
                                              jnp.float32)
                go = lax.bitcast_convert_type(gv & himask, jnp.float32)
                s0 = pl.ds(t * _LANES, _LANES)
                s1 = pl.ds(half + t * _LANES, _LANES)
                rr[e, s0] = rr[e, s0] * ge
                rr[e, s1] = rr[e, s1] * go

    def _issue_scatter(c, islot, slot):
        pltpu.async_copy(rows_ref.at[slot], agg_ref.at[dstb_ref.at[islot]],
                         ss[slot], add=True)

    _sync_idx(0, 0)
    _sync_idx(1, 1)
    _issue_gather(0, 0, 0)

    def _quad(p, carry):
        for k in range(4):
            c = 4 * p + k
            slot = k % 2
            if k == 0:
                @pl.when(p >= 1)
                def _():
                    _wait_scatter(1)
                    _wait_idx(1)
            else:
                _wait_scatter(1 - slot)
                _wait_idx((k + 1) % 4)
            _issue_gather(c + 1, (k + 1) % 4, 1 - slot)
            _wait_gather(slot)
            cc = jnp.minimum(c + 2, nchunks - 1)
            _issue_idx(cc, (k + 2) % 4)
            _compute(slot)
            _issue_scatter(c, k, slot)
        return carry

    lax.fori_loop(0, (nchunks - 1) // 4, _quad, 0)
    _wait_scatter(1)
    _wait_idx(1)
    _wait_gather(0)
    _compute(0)
    _issue_scatter(nchunks - 1, 0, 0)
    _wait_scatter(0)
    plsc.subcore_barrier()

    for t in range(nfull):
        r0 = sid * rows_per_tile + t * _CHUNK
        pltpu.sync_copy(agg_ref.at[pl.ds(r0, _CHUNK), :], zb)
        pltpu.sync_copy(zb, out_hbm.at[cid, pl.ds(r0, _CHUNK), :])
    r0 = sid * rows_per_tile + nfull * _CHUNK
    pltpu.sync_copy(agg_ref.at[pl.ds(r0, tail), :], zb.at[pl.ds(0, tail), :])
    pltpu.sync_copy(zb.at[pl.ds(0, tail), :],
                    out_hbm.at[cid, pl.ds(r0, tail), :])


def _post_body(pp_ref, h_ref, wagg_ref, wself_ref, wp_ref, bp_ref, batch_ref,
               o_ref):
    i = pl.program_id(0)
    agg = pp_ref[0] + pp_ref[1]
    h2 = jnp.maximum(
        jnp.dot(agg, wagg_ref[...], preferred_element_type=jnp.float32)
        + jnp.dot(h_ref[...], wself_ref[...],
                  preferred_element_type=jnp.float32),
        0.0,
    )
    p = jnp.dot(h2, wp_ref[...], preferred_element_type=jnp.float32) \
        + bp_ref[0, 0]
    b_row = batch_ref[0]
    gids_t = lax.broadcasted_iota(jnp.int32, (64, b_row.shape[1]), 0)
    onehot_t = (gids_t == b_row).astype(jnp.float32)

    @pl.when(i == 0)
    def _init():
        o_ref[...] = jnp.zeros_like(o_ref)

    o_ref[...] += jnp.dot(onehot_t, p, preferred_element_type=jnp.float32)


def kernel(x, edge_index, edge_attr, batch, W_emb, b_emb, W_edge, W_self,
           W_agg, W_prop, b_prop):
    N, D = x.shape
    E = edge_index.shape[1]
    G = 64
    BLK = 1000
    nblk = N // BLK
    BLKE = 4000
    nblke = E // BLKE

    h = pl.pallas_call(
        _emb_body,
        grid=(nblk,),
        in_specs=[
            pl.BlockSpec((BLK, D), lambda i: (i, 0)),
            pl.BlockSpec((D, D), lambda i: (0, 0)),
            pl.BlockSpec((1, D), lambda i: (0, 0)),
        ],
        out_specs=pl.BlockSpec((BLK, D), lambda i: (i, 0)),
        out_shape=jax.ShapeDtypeStruct((N, D), jnp.float32),
    )(x, W_emb, b_emb.reshape(1, D))

    gate = pl.pallas_call(
        _gate_body,
        grid=(nblke,),
        in_specs=[
            pl.BlockSpec((BLKE, 4), lambda i: (i, 0)),
            pl.BlockSpec((4, D), lambda i: (0, 0)),
        ],
        out_specs=pl.BlockSpec((BLKE, D // 2), lambda i: (i, 0)),
        out_shape=jax.ShapeDtypeStruct((E, D // 2), jnp.int32),
    )(edge_attr, W_edge)

    edges_per_tile = E // (_NC * _NS)
    nchunks = edges_per_tile // _CHUNK
    partials = pl.kernel(
        _edge_sc_body,
        out_type=jax.ShapeDtypeStruct((_NC, _NPAD, D), jnp.float32),
        mesh=plsc.VectorSubcoreMesh(
            core_axis_name="c", subcore_axis_name="s",
            num_cores=_NC, num_subcores=_NS),
        scratch_types=[
            pltpu.VMEM((4, _CHUNK), jnp.int32),
            pltpu.VMEM((4, _CHUNK), jnp.int32),
            pltpu.VMEM((2, _CHUNK, D), jnp.float32),
            pltpu.VMEM((2, _CHUNK, D // 2), jnp.int32),
            pltpu.VMEM_SHARED((_NPAD, D), jnp.float32),
        ] + [pltpu.SemaphoreType.DMA] * 10,
    )(h, edge_index[0], edge_index[1], gate)

    batch3 = batch.reshape(nblk, 1, BLK)
    graph_out = pl.pallas_call(
        _post_body,
        grid=(nblk,),
        in_specs=[
            pl.BlockSpec((_NC, BLK, D), lambda i: (0, i, 0)),
            pl.BlockSpec((BLK, D), lambda i: (i, 0)),
            pl.BlockSpec((D, D), lambda i: (0, 0)),
            pl.BlockSpec((D, D), lambda i: (0, 0)),
            pl.BlockSpec((D, 1), lambda i: (0, 0)),
            pl.BlockSpec((1, 1), lambda i: (0, 0)),
            pl.BlockSpec((1, 1, BLK), lambda i: (i, 0, 0)),
        ],
        out_specs=pl.BlockSpec((G, 1), lambda i: (0, 0)),
        out_shape=jax.ShapeDtypeStruct((G, 1), jnp.float32),
    )(partials, h, W_agg, W_self, W_prop, b_prop.reshape(1, 1), batch3)

    return jnp.squeeze(graph_out, axis=1)


# speedup vs baseline: 1.2257x; 1.2257x over previous
import jax
import jax.numpy as jnp
from jax import lax
from jax.experimental import pallas as pl
from jax.experimental.pallas import tpu as pltpu
from jax.experimental.pallas import tpu_sc as plsc

_NC = 2
_NS = 16
_LANES = 16
_CHUNK = 80
_NPAD = 10112


def _emb_body(x_ref, w_ref, b_ref, o_ref):
    o_ref[...] = (
        jnp.dot(x_ref[...], w_ref[...], preferred_element_type=jnp.float32)
        + b_ref[...]
    )


def _gate_body(ea_ref, w_ref, o_ref):
    z = jnp.dot(ea_ref[...], w_ref[...], preferred_element_type=jnp.float32)
    gate = 0.5 + 0.5 * jnp.tanh(0.5 * z)
    d2 = gate.shape[1] // 2
    a = gate[:, :d2].astype(jnp.bfloat16).astype(jnp.float32)
    b = gate[:, d2:].astype(jnp.bfloat16).astype(jnp.float32)
    ai = lax.bitcast_convert_type(a, jnp.int32)
    bi = lax.bitcast_convert_type(b, jnp.int32)
    o_ref[...] = jnp.bitwise_or(lax.shift_right_logical(ai, 16), bi)


def _edge_sc_body(h_hbm, ei_hbm, gate_hbm, out_hbm,
                  srcb_ref, dstb_ref, rows_ref, gbuf_ref,
                  agg_ref, sg0, sg1, sa0, sa1, ss0, ss1,
                  si0, si1, si2, si3):
    N, D = h_hbm.shape
    E = ei_hbm.shape[0] // 2
    nj = D // _LANES
    cid = lax.axis_index("c")
    sid = lax.axis_index("s")
    rows_per_tile = _NPAD // _NS
    tile_id = cid * _NS + sid
    edges_per_tile = E // (_NC * _NS)
    nchunks = edges_per_tile // _CHUNK
    ebase = tile_id * edges_per_tile
    sg = (sg0, sg1)
    sa = (sa0, sa1)
    ss = (ss0, ss1)
    si = (si0, si1, si2, si3)

    def _idx_slices(c):
        return (ei_hbm.at[pl.ds(ebase + c * _CHUNK, _CHUNK)],
                ei_hbm.at[pl.ds(E + ebase + c * _CHUNK, _CHUNK)])

    def _issue_idx(c, slot):
        s, d = _idx_slices(c)
        pltpu.async_copy(s, srcb_ref.at[slot], si[slot])
        pltpu.async_copy(d, dstb_ref.at[slot], si[slot])

    def _wait_idx(slot):
        pltpu.make_async_copy(ei_hbm.at[pl.ds(0, _CHUNK)],
                              srcb_ref.at[slot], si[slot]).wait()
        pltpu.make_async_copy(ei_hbm.at[pl.ds(0, _CHUNK)],
                              dstb_ref.at[slot], si[slot]).wait()

    def _sync_idx(c, slot):
        s, d = _idx_slices(c)
        pltpu.sync_copy(s, srcb_ref.at[slot])
        pltpu.sync_copy(d, dstb_ref.at[slot])

    zb = rows_ref.at[0]
    zeros = jnp.zeros((_LANES,), jnp.float32)

    def _zrow(r, carry):
        for j in range(nj):
            zb[r, pl.ds(j * _LANES, _LANES)] = zeros
        return carry

    lax.fori_loop(0, _CHUNK, _zrow, 0)
    nfull = rows_per_tile // _CHUNK
    tail = rows_per_tile - nfull * _CHUNK
    for t in range(nfull):
        r0 = sid * rows_per_tile + t * _CHUNK
        pltpu.sync_copy(zb, agg_ref.at[pl.ds(r0, _CHUNK), :])
    pltpu.sync_copy(zb.at[pl.ds(0, tail), :],
                    agg_ref.at[pl.ds(sid * rows_per_tile + nfull * _CHUNK,
                                     tail), :])
    plsc.subcore_barrier()

    def _issue_gather(c, islot, slot):
        pltpu.async_copy(h_hbm.at[srcb_ref.at[islot]],
                         rows_ref.at[slot], sg[slot])
        pltpu.async_copy(
            gate_hbm.at[pl.ds(ebase + c * _CHUNK, _CHUNK), :],
            gbuf_ref.at[slot], sa[slot])

    def _wait_gather(slot):
        pltpu.make_async_copy(h_hbm.at[pl.ds(0, _CHUNK)],
                              rows_ref.at[slot], sg[slot]).wait()
        pltpu.make_async_copy(gate_hbm.at[pl.ds(0, _CHUNK), :],
                              gbuf_ref.at[slot], sa[slot]).wait()

    def _wait_scatter(slot):
        pltpu.make_async_copy(rows_ref.at[slot],
                              agg_ref.at[pl.ds(0, _CHUNK), :],
                              ss[slot]).wait()

    def _compute(slot):
        rr = rows_ref.at[slot]
        gg = gbuf_ref.at[slot]
        himask = jnp.int32(-65536)
        half = (nj // 2) * _LANES

        @plsc.parallel_loop(0, _CHUNK, unroll=2)
        def _e(e):
            for t in range(nj // 2):
                gv = gg[e, pl.ds(t * _LANES, _LANES)]
                ge = lax.bitcast_convert_type(jnp.left_shift(gv, 16),
                                              jnp.float32)
                go = lax.bitcast_convert_type(gv & himask, jnp.float32)
                s0 = pl.ds(t * _LANES, _LANES)
                s1 = pl.ds(half + t * _LANES, _LANES)
                rr[e, s0] = rr[e, s0] * ge
                rr[e, s1] = rr[e, s1] * go

    def _issue_scatter(c, islot, slot):
        pltpu.async_copy(rows_ref.at[slot], agg_ref.at[dstb_ref.at[islot]],
                         ss[slot], add=True)

    _sync_idx(0, 0)
    _sync_idx(1, 1)
    _issue_gather(0, 0, 0)

    def _quad(p, carry):
        for k in range(4):
            c = 4 * p + k
            slot = k % 2
            if k == 0:
                @pl.when(p >= 1)
                def _():
                    _wait_scatter(1)
                    _wait_idx(1)
            else:
                _wait_scatter(1 - slot)
                _wait_idx((k + 1) % 4)
            _issue_gather(c + 1, (k + 1) % 4, 1 - slot)
            _wait_gather(slot)
            cc = jnp.minimum(c + 2, nchunks - 1)
            _issue_idx(cc, (k + 2) % 4)
            _compute(slot)
            _issue_scatter(c, k, slot)
        return carry

    lax.fori_loop(0, (nchunks - 1) // 4, _quad, 0)
    _wait_scatter(1)
    _wait_idx(1)
    _wait_gather(0)
    _compute(0)
    _issue_scatter(nchunks - 1, 0, 0)
    _wait_scatter(0)
    plsc.subcore_barrier()

    for t in range(nfull):
        r0 = sid * rows_per_tile + t * _CHUNK
        pltpu.sync_copy(agg_ref.at[pl.ds(r0, _CHUNK), :], zb)
        pltpu.sync_copy(zb, out_hbm.at[cid, pl.ds(r0, _CHUNK), :])
    r0 = sid * rows_per_tile + nfull * _CHUNK
    pltpu.sync_copy(agg_ref.at[pl.ds(r0, tail), :], zb.at[pl.ds(0, tail), :])
    pltpu.sync_copy(zb.at[pl.ds(0, tail), :],
                    out_hbm.at[cid, pl.ds(r0, tail), :])


def _post_body(pp_ref, h_ref, wagg_ref, wself_ref, wp_ref, bp_ref, batch_ref,
               o_ref):
    i = pl.program_id(0)
    agg = pp_ref[0] + pp_ref[1]
    h2 = jnp.maximum(
        jnp.dot(agg, wagg_ref[...], preferred_element_type=jnp.float32)
        + jnp.dot(h_ref[...], wself_ref[...],
                  preferred_element_type=jnp.float32),
        0.0,
    )
    p = jnp.dot(h2, wp_ref[...], preferred_element_type=jnp.float32) \
        + bp_ref[0, 0]
    b_row = batch_ref[0]
    gids_t = lax.broadcasted_iota(jnp.int32, (64, b_row.shape[1]), 0)
    onehot_t = (gids_t == b_row).astype(jnp.float32)

    @pl.when(i == 0)
    def _init():
        o_ref[...] = jnp.zeros_like(o_ref)

    o_ref[...] += jnp.dot(onehot_t, p, preferred_element_type=jnp.float32)


def kernel(x, edge_index, edge_attr, batch, W_emb, b_emb, W_edge, W_self,
           W_agg, W_prop, b_prop):
    N, D = x.shape
    E = edge_index.shape[1]
    G = 64
    BLK = 1000
    nblk = N // BLK
    BLKE = 4000
    nblke = E // BLKE

    h = pl.pallas_call(
        _emb_body,
        grid=(nblk,),
        in_specs=[
            pl.BlockSpec((BLK, D), lambda i: (i, 0)),
            pl.BlockSpec((D, D), lambda i: (0, 0)),
            pl.BlockSpec((1, D), lambda i: (0, 0)),
        ],
        out_specs=pl.BlockSpec((BLK, D), lambda i: (i, 0)),
        out_shape=jax.ShapeDtypeStruct((N, D), jnp.float32),
    )(x, W_emb, b_emb.reshape(1, D))

    gate = pl.pallas_call(
        _gate_body,
        grid=(nblke,),
        in_specs=[
            pl.BlockSpec((BLKE, 4), lambda i: (i, 0)),
            pl.BlockSpec((4, D), lambda i: (0, 0)),
        ],
        out_specs=pl.BlockSpec((BLKE, D // 2), lambda i: (i, 0)),
        out_shape=jax.ShapeDtypeStruct((E, D // 2), jnp.int32),
    )(edge_attr, W_edge)

    edges_per_tile = E // (_NC * _NS)
    nchunks = edges_per_tile // _CHUNK
    partials = pl.kernel(
        _edge_sc_body,
        out_type=jax.ShapeDtypeStruct((_NC, _NPAD, D), jnp.float32),
        mesh=plsc.VectorSubcoreMesh(
            core_axis_name="c", subcore_axis_name="s",
            num_cores=_NC, num_subcores=_NS),
        scratch_types=[
            pltpu.VMEM((4, _CHUNK), jnp.int32),
            pltpu.VMEM((4, _CHUNK), jnp.int32),
            pltpu.VMEM((2, _CHUNK, D), jnp.float32),
            pltpu.VMEM((2, _CHUNK, D // 2), jnp.int32),
            pltpu.VMEM_SHARED((_NPAD, D), jnp.float32),
        ] + [pltpu.SemaphoreType.DMA] * 10,
    )(h, edge_index.reshape(2 * E), gate)

    batch3 = batch.reshape(nblk, 1, BLK)
    graph_out = pl.pallas_call(
        _post_body,
        grid=(nblk,),
        in_specs=[
            pl.BlockSpec((_NC, BLK, D), lambda i: (0, i, 0)),
            pl.BlockSpec((BLK, D), lambda i: (i, 0)),
            pl.BlockSpec((D, D), lambda i: (0, 0)),
            pl.BlockSpec((D, D), lambda i: (0, 0)),
            pl.BlockSpec((D, 1), lambda i: (0, 0)),
            pl.BlockSpec((1, 1), lambda i: (0, 0)),
            pl.BlockSpec((1, 1, BLK), lambda i: (i, 0, 0)),
        ],
        out_specs=pl.BlockSpec((G, 1), lambda i: (0, 0)),
        out_shape=jax.ShapeDtypeStruct((G, 1), jnp.float32),
    )(partials, h, W_agg, W_self, W_prop, b_prop.reshape(1, 1), batch3)

    return jnp.squeeze(graph_out, axis=1)
